# Initial kernel scaffold; baseline (speedup 1.0000x reference)
import jax
import jax.numpy as jnp
from jax.experimental import pallas as pl


def kernel(x):
    raise NotImplementedError("write your pallas kernel here")


# speedup vs baseline: 1.2930x; 1.2930x over previous
import functools

import jax
import jax.numpy as jnp
from jax.experimental import pallas as pl
from jax.experimental.pallas import tpu as pltpu


def _mean_rows_kernel(x_ref, o_ref, *, inv_r):
    x = x_ref[...]
    o_ref[...] = jnp.sum(x, axis=-1, keepdims=True) * inv_r


def kernel(x):
    N, C, H, W = x.shape
    M = N * C
    R = H * W
    x2 = x.reshape(M, R)

    TM = 8192
    grid = (M // TM,)

    out = pl.pallas_call(
        functools.partial(_mean_rows_kernel, inv_r=1.0 / R),
        out_shape=jax.ShapeDtypeStruct((M, 1), x.dtype),
        grid=grid,
        in_specs=[pl.BlockSpec((TM, R), lambda i: (i, 0))],
        out_specs=pl.BlockSpec((TM, 1), lambda i: (i, 0)),
        compiler_params=pltpu.CompilerParams(
            dimension_semantics=("parallel",),
            vmem_limit_bytes=64 * 1024 * 1024,
        ),
        cost_estimate=pl.CostEstimate(
            flops=M * R,
            transcendentals=0,
            bytes_accessed=M * R * 4 + M * 4,
        ),
    )(x2)
    return out.reshape(N, C)
